# Initial kernel scaffold; baseline (speedup 1.0000x reference)
import jax
import jax.numpy as jnp
from jax.experimental import pallas as pl


def kernel(u2i_src, u2i_dst, i2u_src, i2u_dst, emb_user, emb_item, W1un, b1un, W1us, b1us, W1in, b1in, W1is, b1is, W2un, b2un, W2us, b2us, W2in, b2in, W2is, b2is, W_head, b_head):
    raise NotImplementedError("write your pallas kernel here")


# speedup vs baseline: 2.3466x; 2.3466x over previous
import functools

import jax
import jax.numpy as jnp
from jax import lax
from jax.experimental import pallas as pl
from jax.experimental.pallas import tpu as pltpu
from jax.experimental.pallas import tpu_sc as plsc

N = 10000
E = 320000
D = 128
H = 64
NC = 2
NS = 16
CH = 80
SLAB = 624
ZROWS = 104


def _fill_f32(ref, nrows, ncols, value):
    per_row = ncols // 16

    def body(t, carry):
        ref[t // per_row, pl.ds((t % per_row) * 16, 16)] = jnp.full(
            (16,), value, jnp.float32)
        return carry

    lax.fori_loop(0, nrows * per_row, body, None)


def _make_seg_sum(n_dirs, chunks_per_dir):
    nch = n_dirs * chunks_per_dir
    mesh = plsc.VectorSubcoreMesh(core_axis_name="c", subcore_axis_name="s")

    @functools.partial(
        pl.kernel,
        out_type=jax.ShapeDtypeStruct((n_dirs, NC, N, H), jnp.float32),
        mesh=mesh,
        scratch_types=[
            pltpu.VMEM((CH,), jnp.int32),
            pltpu.VMEM((CH,), jnp.int32),
            pltpu.VMEM((CH, H), jnp.float32),
            pltpu.VMEM((ZROWS, H), jnp.float32),
            pltpu.VMEM_SHARED((N, H), jnp.float32),
            pltpu.SemaphoreType.DMA,
        ],
        compiler_params=pltpu.CompilerParams(use_tc_tiling_on_sc=False),
    )
    def seg_sum(table_hbm, src_hbm, dst_hbm, out_hbm,
                src_v, dst_v, rows_v, zero_v, acc_sh, sem):
        c = lax.axis_index("c")
        s = lax.axis_index("s")
        _fill_f32(zero_v, ZROWS, H, 0.0)

        def dir_body(d, carry):
            for r in range(SLAB // ZROWS):
                pltpu.sync_copy(zero_v,
                                acc_sh.at[pl.ds(s * SLAB + r * ZROWS, ZROWS)])

            @pl.when(s == NS - 1)
            def _zero_tail():
                pltpu.sync_copy(zero_v.at[pl.ds(0, N - NS * SLAB)],
                                acc_sh.at[pl.ds(NS * SLAB, N - NS * SLAB)])

            plsc.subcore_barrier()

            def body(k, inner):
                pltpu.sync_copy(src_hbm.at[c, s, k], src_v)
                pltpu.sync_copy(dst_hbm.at[c, s, k], dst_v)
                pltpu.async_copy(table_hbm.at[src_v], rows_v, sem).wait()
                pltpu.sync_copy(rows_v, acc_sh.at[dst_v], add=True)
                return inner

            lax.fori_loop(d * chunks_per_dir, (d + 1) * chunks_per_dir,
                          body, None)
            plsc.subcore_barrier()

            @pl.when(s == 0)
            def _write_out():
                pltpu.sync_copy(acc_sh, out_hbm.at[d, c])

            plsc.subcore_barrier()
            return carry

        lax.fori_loop(0, n_dirs, dir_body, None)

    return seg_sum


def _make_counts(n_chunks):
    mesh = plsc.VectorSubcoreMesh(core_axis_name="c", subcore_axis_name="s")

    @functools.partial(
        pl.kernel,
        out_type=jax.ShapeDtypeStruct((NC, N, 16), jnp.float32),
        mesh=mesh,
        scratch_types=[
            pltpu.VMEM((CH,), jnp.int32),
            pltpu.VMEM((CH, 16), jnp.float32),
            pltpu.VMEM((ZROWS, 16), jnp.float32),
            pltpu.VMEM_SHARED((N, 16), jnp.float32),
        ],
        compiler_params=pltpu.CompilerParams(use_tc_tiling_on_sc=False),
    )
    def counts(dst_hbm, out_hbm, dst_v, ones_v, zero_v, acc_sh):
        c = lax.axis_index("c")
        s = lax.axis_index("s")
        _fill_f32(ones_v, CH, 16, 1.0)
        _fill_f32(zero_v, ZROWS, 16, 0.0)
        for r in range(SLAB // ZROWS):
            pltpu.sync_copy(zero_v,
                            acc_sh.at[pl.ds(s * SLAB + r * ZROWS, ZROWS)])

        @pl.when(s == NS - 1)
        def _zero_tail():
            pltpu.sync_copy(zero_v.at[pl.ds(0, N - NS * SLAB)],
                            acc_sh.at[pl.ds(NS * SLAB, N - NS * SLAB)])

        plsc.subcore_barrier()

        def body(k, carry):
            pltpu.sync_copy(dst_hbm.at[c, s, k], dst_v)
            pltpu.sync_copy(ones_v, acc_sh.at[dst_v], add=True)
            return carry

        lax.fori_loop(0, n_chunks, body, None)
        plsc.subcore_barrier()

        @pl.when(s == 0)
        def _write_out():
            pltpu.sync_copy(acc_sh, out_hbm.at[c])

    return counts


_RB = 1000
_NB = N // _RB


def _relu_body(x_ref, o_ref):
    o_ref[...] = jnp.maximum(x_ref[...], 0.0)


def _relu(x):
    n = x.shape[0]
    return pl.pallas_call(
        _relu_body,
        grid=(n // _RB,),
        in_specs=[pl.BlockSpec((_RB, D), lambda j: (j, 0))],
        out_specs=pl.BlockSpec((_RB, D), lambda j: (j, 0)),
        out_shape=jax.ShapeDtypeStruct((n, D), jnp.float32),
    )(x)


def _conv_pair_body(alo_ref, ahi_ref, cnt_ref, x_ref, w_ref, b_ref, o_ref):
    r = 1.0 / jnp.maximum(cnt_ref[0][:, 0:1], 1.0)
    a = jnp.concatenate([alo_ref[0, 0], ahi_ref[0, 0]], axis=1) * r
    o = (jnp.dot(a, w_ref[0, :D], preferred_element_type=jnp.float32)
         + jnp.dot(x_ref[...], w_ref[0, D:],
                   preferred_element_type=jnp.float32)
         + b_ref[0])
    o_ref[0] = jnp.maximum(o, 0.0)


def _conv_pair(agg, cnt, x_cat, w_cat, b_cat):
    return pl.pallas_call(
        _conv_pair_body,
        grid=(2, _NB),
        in_specs=[
            pl.BlockSpec((1, 1, _RB, H), lambda d, j: (d, 0, j, 0)),
            pl.BlockSpec((1, 1, _RB, H), lambda d, j: (d, 1, j, 0)),
            pl.BlockSpec((1, _RB, 16), lambda d, j: (d, j, 0)),
            pl.BlockSpec((_RB, D), lambda d, j: ((1 - d) * _NB + j, 0)),
            pl.BlockSpec((1, 2 * D, D), lambda d, j: (d, 0, 0)),
            pl.BlockSpec((1, 1, D), lambda d, j: (d, 0, 0)),
        ],
        out_specs=pl.BlockSpec((1, _RB, D), lambda d, j: (d, j, 0)),
        out_shape=jax.ShapeDtypeStruct((2, N, D), jnp.float32),
    )(agg, agg, cnt, x_cat, w_cat, b_cat)


def _final_body(plo_ref, phi_ref, cnt_ref, x_ref, w_ref, b_ref,
                wh_ref, bh_ref, o_ref):
    r = 1.0 / jnp.maximum(cnt_ref[0][:, 0:1], 1.0)
    a = jnp.concatenate([plo_ref[0, 0], phi_ref[0, 0]], axis=1) * r
    o = (jnp.dot(a, w_ref[:D], preferred_element_type=jnp.float32)
         + jnp.dot(x_ref[0], w_ref[D:], preferred_element_type=jnp.float32)
         + b_ref[...])
    o = jnp.maximum(o, 0.0)
    o_ref[...] = jnp.sum(o * wh_ref[...], axis=1, keepdims=True) + bh_ref[...]


def _final(agg2, cnts, x2, w_cat, b_cat, wh_row, bh):
    return pl.pallas_call(
        _final_body,
        grid=(_NB,),
        in_specs=[
            pl.BlockSpec((1, 1, _RB, H), lambda j: (0, 0, j, 0)),
            pl.BlockSpec((1, 1, _RB, H), lambda j: (0, 1, j, 0)),
            pl.BlockSpec((1, _RB, 16), lambda j: (0, j, 0)),
            pl.BlockSpec((1, _RB, D), lambda j: (0, j, 0)),
            pl.BlockSpec((2 * D, D), lambda j: (0, 0)),
            pl.BlockSpec((1, D), lambda j: (0, 0)),
            pl.BlockSpec((1, D), lambda j: (0, 0)),
            pl.BlockSpec((1, 1), lambda j: (0, 0)),
        ],
        out_specs=pl.BlockSpec((_RB, 1), lambda j: (j, 0)),
        out_shape=jax.ShapeDtypeStruct((N, 1), jnp.float32),
    )(agg2, agg2, cnts, x2, w_cat, b_cat, wh_row, bh)


def kernel(u2i_src, u2i_dst, i2u_src, i2u_dst, emb_user, emb_item,
           W1un, b1un, W1us, b1us, W1in, b1in, W1is, b1is,
           W2un, b2un, W2us, b2us, W2in, b2in, W2is, b2is, W_head, b_head):
    ncd = E // (NS * CH)

    def chunked(idx):
        return jnp.broadcast_to(
            idx.reshape(1, NS, ncd, CH).astype(jnp.int32),
            (NC, NS, ncd, CH))

    core_off = jnp.arange(NC, dtype=jnp.int32).reshape(NC, 1, 1, 1)
    src1 = jnp.concatenate(
        [2 * chunked(u2i_src) + core_off,
         2 * chunked(i2u_src + N) + core_off], axis=2)
    dst1 = jnp.concatenate([chunked(u2i_dst), chunked(i2u_dst)], axis=2)
    src2 = 2 * chunked(u2i_src + N) + core_off
    dst2 = chunked(u2i_dst)

    x_cat = _relu(jnp.concatenate([emb_user, emb_item], axis=0))

    dst_cnt = jnp.stack([chunked(u2i_dst)[0], chunked(i2u_dst)[0]])
    cnts = _make_counts(ncd)(dst_cnt)

    agg1 = _make_seg_sum(2, ncd)(x_cat.reshape(4 * N, H), src1, dst1)

    w1 = jnp.stack([jnp.concatenate([W1un, W1us], axis=0),
                    jnp.concatenate([W1in, W1is], axis=0)])
    b1 = jnp.stack([(b1un + b1us)[None, :], (b1in + b1is)[None, :]])
    x2 = _conv_pair(agg1, cnts, x_cat, w1, b1)

    agg2 = _make_seg_sum(1, ncd)(x2.reshape(4 * N, H), src2, dst2)

    w2 = jnp.concatenate([W2un, W2us], axis=0)
    b2 = (b2un + b2us)[None, :]
    out = _final(agg2, cnts, x2, w2, b2, W_head.T, b_head[None, :])
    return out[:, 0]


# speedup vs baseline: 4.9652x; 2.1159x over previous
import functools

import jax
import jax.numpy as jnp
from jax import lax
from jax.experimental import pallas as pl
from jax.experimental.pallas import tpu as pltpu
from jax.experimental.pallas import tpu_sc as plsc

N = 10000
E = 320000
D = 128
H = 64
NC = 2
NS = 16
CH = 80
SLAB = 624
ZR = 16
NBUF = 3

_SC_PARAMS = pltpu.CompilerParams(use_tc_tiling_on_sc=False)


def _fill_f32(ref, nrows, ncols, value):
    per_row = ncols // 16

    def body(t, carry):
        ref[t // per_row, pl.ds((t % per_row) * 16, 16)] = jnp.full(
            (16,), value, jnp.float32)
        return carry

    lax.fori_loop(0, nrows * per_row, body, None)


def _zero_acc(acc_sh, zero_v, s):
    for r in range(SLAB // ZR):
        pltpu.sync_copy(zero_v, acc_sh.at[pl.ds(s * SLAB + r * ZR, ZR)])

    @pl.when(s == NS - 1)
    def _zero_tail():
        pltpu.sync_copy(zero_v.at[pl.ds(0, N - NS * SLAB)],
                        acc_sh.at[pl.ds(NS * SLAB, N - NS * SLAB)])


def _make_seg_sum(n_dirs, ncd):
    mesh = plsc.VectorSubcoreMesh(core_axis_name="c", subcore_axis_name="s")

    @functools.partial(
        pl.kernel,
        out_type=jax.ShapeDtypeStruct((n_dirs, NC, N, H), jnp.float32),
        mesh=mesh,
        scratch_types=[
            [pltpu.VMEM((2, CH), jnp.int32)] * NBUF,
            [pltpu.VMEM((CH, H), jnp.float32)] * NBUF,
            pltpu.VMEM((ZR, H), jnp.float32),
            pltpu.VMEM_SHARED((N, H), jnp.float32),
            [pltpu.SemaphoreType.DMA] * NBUF,
            [pltpu.SemaphoreType.DMA] * NBUF,
        ],
        compiler_params=_SC_PARAMS,
    )
    def seg_sum(table_hbm, idx_hbm, out_hbm,
                idx_v, rows_v, zero_v, acc_sh, isem, gsem):
        c = lax.axis_index("c")
        s = lax.axis_index("s")
        _fill_f32(zero_v, ZR, H, 0.0)

        def load_idx(k, b):
            pltpu.async_copy(idx_hbm.at[c, s, k], idx_v[b], isem[b])

        def wait_idx(k, b):
            pltpu.make_async_copy(idx_hbm.at[c, s, k], idx_v[b],
                                  isem[b]).wait()

        def start_gather(b):
            pltpu.async_copy(table_hbm.at[idx_v[b].at[0]], rows_v[b],
                             gsem[b])

        def wait_gather(b):
            pltpu.make_async_copy(table_hbm.at[idx_v[b].at[0]], rows_v[b],
                                  gsem[b]).wait()

        def dir_body(d, carry):
            base = d * ncd
            for b in range(NBUF):
                load_idx(base + b, b)
            for b in range(NBUF - 1):
                wait_idx(base + b, b)
                start_gather(b)
            _zero_acc(acc_sh, zero_v, s)
            plsc.subcore_barrier()

            def tri(t, inner):
                for b in range(NBUF):
                    j = NBUF * t + b

                    b2 = (b + NBUF - 1) % NBUF

                    @pl.when(j + NBUF - 1 < ncd)
                    def _issue_ahead():
                        wait_idx(base + j + NBUF - 1, b2)
                        start_gather(b2)

                    @pl.when(j < ncd)
                    def _consume():
                        wait_gather(b)
                        pltpu.sync_copy(rows_v[b],
                                        acc_sh.at[idx_v[b].at[1]],
                                        add=True)

                    @pl.when(j + NBUF < ncd)
                    def _prefetch():
                        load_idx(base + j + NBUF, b)
                return inner

            lax.fori_loop(0, (ncd + NBUF - 1) // NBUF, tri, None)
            plsc.subcore_barrier()

            @pl.when(s == 0)
            def _write_out():
                pltpu.sync_copy(acc_sh, out_hbm.at[d, c])

            plsc.subcore_barrier()
            return carry

        lax.fori_loop(0, n_dirs, dir_body, None)

    return seg_sum


def _make_counts(ncd):
    mesh = plsc.VectorSubcoreMesh(core_axis_name="c", subcore_axis_name="s")

    @functools.partial(
        pl.kernel,
        out_type=jax.ShapeDtypeStruct((NC, N, 16), jnp.float32),
        mesh=mesh,
        scratch_types=[
            [pltpu.VMEM((CH,), jnp.int32)] * 2,
            pltpu.VMEM((CH, 16), jnp.float32),
            pltpu.VMEM((ZR, 16), jnp.float32),
            pltpu.VMEM_SHARED((N, 16), jnp.float32),
            [pltpu.SemaphoreType.DMA] * 2,
        ],
        compiler_params=_SC_PARAMS,
    )
    def counts(dst_hbm, out_hbm, dst_v, ones_v, zero_v, acc_sh, isem):
        c = lax.axis_index("c")
        s = lax.axis_index("s")

        def load_idx(k, b):
            pltpu.async_copy(dst_hbm.at[c, s, k], dst_v[b], isem[b])

        def wait_idx(k, b):
            pltpu.make_async_copy(dst_hbm.at[c, s, k], dst_v[b],
                                  isem[b]).wait()

        for b in range(2):
            load_idx(b, b)
        _fill_f32(ones_v, CH, 16, 1.0)
        _fill_f32(zero_v, ZR, 16, 0.0)
        for r in range(SLAB // ZR):
            pltpu.sync_copy(zero_v, acc_sh.at[pl.ds(s * SLAB + r * ZR, ZR)])

        @pl.when(s == NS - 1)
        def _zero_tail():
            pltpu.sync_copy(zero_v.at[pl.ds(0, N - NS * SLAB)],
                            acc_sh.at[pl.ds(NS * SLAB, N - NS * SLAB)])

        plsc.subcore_barrier()

        def duo(t, carry):
            for b in range(2):
                j = 2 * t + b

                @pl.when(j < ncd)
                def _consume():
                    wait_idx(j, b)
                    pltpu.sync_copy(ones_v, acc_sh.at[dst_v[b]], add=True)

                @pl.when(j + 2 < ncd)
                def _prefetch():
                    load_idx(j + 2, b)
            return carry

        lax.fori_loop(0, (ncd + 1) // 2, duo, None)
        plsc.subcore_barrier()

        @pl.when(s == 0)
        def _write_out():
            pltpu.sync_copy(acc_sh, out_hbm.at[c])

    return counts


_RB = 1000
_NB = N // _RB


def _relu_body(x_ref, o_ref):
    o_ref[...] = jnp.maximum(x_ref[...], 0.0)


def _relu(x):
    n = x.shape[0]
    return pl.pallas_call(
        _relu_body,
        grid=(n // _RB,),
        in_specs=[pl.BlockSpec((_RB, D), lambda j: (j, 0))],
        out_specs=pl.BlockSpec((_RB, D), lambda j: (j, 0)),
        out_shape=jax.ShapeDtypeStruct((n, D), jnp.float32),
    )(x)


def _conv_pair_body(alo_ref, ahi_ref, cnt_ref, x_ref, w_ref, b_ref, o_ref):
    r = 1.0 / jnp.maximum(cnt_ref[0][:, 0:1], 1.0)
    a = jnp.concatenate([alo_ref[0, 0], ahi_ref[0, 0]], axis=1) * r
    o = (jnp.dot(a, w_ref[0, :D], preferred_element_type=jnp.float32)
         + jnp.dot(x_ref[...], w_ref[0, D:],
                   preferred_element_type=jnp.float32)
         + b_ref[0])
    o_ref[0] = jnp.maximum(o, 0.0)


def _conv_pair(agg, cnt, x_cat, w_cat, b_cat):
    return pl.pallas_call(
        _conv_pair_body,
        grid=(2, _NB),
        in_specs=[
            pl.BlockSpec((1, 1, _RB, H), lambda d, j: (d, 0, j, 0)),
            pl.BlockSpec((1, 1, _RB, H), lambda d, j: (d, 1, j, 0)),
            pl.BlockSpec((1, _RB, 16), lambda d, j: (d, j, 0)),
            pl.BlockSpec((_RB, D), lambda d, j: ((1 - d) * _NB + j, 0)),
            pl.BlockSpec((1, 2 * D, D), lambda d, j: (d, 0, 0)),
            pl.BlockSpec((1, 1, D), lambda d, j: (d, 0, 0)),
        ],
        out_specs=pl.BlockSpec((1, _RB, D), lambda d, j: (d, j, 0)),
        out_shape=jax.ShapeDtypeStruct((2, N, D), jnp.float32),
    )(agg, agg, cnt, x_cat, w_cat, b_cat)


def _final_body(plo_ref, phi_ref, cnt_ref, x_ref, w_ref, b_ref,
                wh_ref, bh_ref, o_ref):
    r = 1.0 / jnp.maximum(cnt_ref[0][:, 0:1], 1.0)
    a = jnp.concatenate([plo_ref[0, 0], phi_ref[0, 0]], axis=1) * r
    o = (jnp.dot(a, w_ref[:D], preferred_element_type=jnp.float32)
         + jnp.dot(x_ref[0], w_ref[D:], preferred_element_type=jnp.float32)
         + b_ref[...])
    o = jnp.maximum(o, 0.0)
    o_ref[...] = jnp.sum(o * wh_ref[...], axis=1, keepdims=True) + bh_ref[...]


def _final(agg2, cnts, x2, w_cat, b_cat, wh_row, bh):
    return pl.pallas_call(
        _final_body,
        grid=(_NB,),
        in_specs=[
            pl.BlockSpec((1, 1, _RB, H), lambda j: (0, 0, j, 0)),
            pl.BlockSpec((1, 1, _RB, H), lambda j: (0, 1, j, 0)),
            pl.BlockSpec((1, _RB, 16), lambda j: (0, j, 0)),
            pl.BlockSpec((1, _RB, D), lambda j: (0, j, 0)),
            pl.BlockSpec((2 * D, D), lambda j: (0, 0)),
            pl.BlockSpec((1, D), lambda j: (0, 0)),
            pl.BlockSpec((1, D), lambda j: (0, 0)),
            pl.BlockSpec((1, 1), lambda j: (0, 0)),
        ],
        out_specs=pl.BlockSpec((_RB, 1), lambda j: (j, 0)),
        out_shape=jax.ShapeDtypeStruct((N, 1), jnp.float32),
    )(agg2, agg2, cnts, x2, w_cat, b_cat, wh_row, bh)


def kernel(u2i_src, u2i_dst, i2u_src, i2u_dst, emb_user, emb_item,
           W1un, b1un, W1us, b1us, W1in, b1in, W1is, b1is,
           W2un, b2un, W2us, b2us, W2in, b2in, W2is, b2is, W_head, b_head):
    ncd = E // (NS * CH)

    def chunked(idx):
        return jnp.broadcast_to(
            idx.reshape(1, NS, ncd, 1, CH).astype(jnp.int32),
            (NC, NS, ncd, 1, CH))

    core_off = jnp.arange(NC, dtype=jnp.int32).reshape(NC, 1, 1, 1, 1)
    src1 = jnp.concatenate(
        [2 * chunked(u2i_src) + core_off,
         2 * chunked(i2u_src + N) + core_off], axis=2)
    dst1 = jnp.concatenate([chunked(u2i_dst), chunked(i2u_dst)], axis=2)
    idx1 = jnp.concatenate([src1, dst1], axis=3)
    idx2 = jnp.concatenate(
        [2 * chunked(u2i_src + N) + core_off, chunked(u2i_dst)], axis=3)

    x_cat = _relu(jnp.concatenate([emb_user, emb_item], axis=0))

    dst_cnt = jnp.stack([chunked(u2i_dst)[0, :, :, 0],
                         chunked(i2u_dst)[0, :, :, 0]])
    cnts = _make_counts(ncd)(dst_cnt)

    agg1 = _make_seg_sum(2, ncd)(x_cat.reshape(4 * N, H), idx1)

    w1 = jnp.stack([jnp.concatenate([W1un, W1us], axis=0),
                    jnp.concatenate([W1in, W1is], axis=0)])
    b1 = jnp.stack([(b1un + b1us)[None, :], (b1in + b1is)[None, :]])
    x2 = _conv_pair(agg1, cnts, x_cat, w1, b1)

    agg2 = _make_seg_sum(1, ncd)(x2.reshape(4 * N, H), idx2)

    w2 = jnp.concatenate([W2un, W2us], axis=0)
    b2 = (b2un + b2us)[None, :]
    out = _final(agg2, cnts, x2, w2, b2, W_head.T, b_head[None, :])
    return out[:, 0]


# speedup vs baseline: 7.4418x; 1.4988x over previous
import functools

import jax
import jax.numpy as jnp
from jax import lax
from jax.experimental import pallas as pl
from jax.experimental.pallas import tpu as pltpu
from jax.experimental.pallas import tpu_sc as plsc

N = 10000
E = 320000
D = 128
H = 64
NC = 2
NS = 16
CH = 80
SLAB = 624
ZR = 16
NBUF = 3

_SC_PARAMS = pltpu.CompilerParams(use_tc_tiling_on_sc=False)


def _fill_f32(ref, nrows, ncols, value):
    per_row = ncols // 16

    def body(t, carry):
        ref[t // per_row, pl.ds((t % per_row) * 16, 16)] = jnp.full(
            (16,), value, jnp.float32)
        return carry

    lax.fori_loop(0, nrows * per_row, body, None)


def _zero_acc(acc_sh, zero_v, s):
    for r in range(SLAB // ZR):
        pltpu.sync_copy(zero_v, acc_sh.at[pl.ds(s * SLAB + r * ZR, ZR)])

    @pl.when(s == NS - 1)
    def _zero_tail():
        pltpu.sync_copy(zero_v.at[pl.ds(0, N - NS * SLAB)],
                        acc_sh.at[pl.ds(NS * SLAB, N - NS * SLAB)])


def _make_seg_sum(n_dirs, ncd, mul, offs, with_counts=False):
    mesh = plsc.VectorSubcoreMesh(core_axis_name="c", subcore_axis_name="s")
    cmul, doffs = offs

    out_type = jax.ShapeDtypeStruct((n_dirs, NC, N, H), jnp.float32)
    if with_counts:
        out_type = (out_type,
                    jax.ShapeDtypeStruct((n_dirs, N, 16), jnp.float32))
    scratch = [
        [pltpu.VMEM((CH,), jnp.int32)] * NBUF,
        [pltpu.VMEM((CH,), jnp.int32)] * NBUF,
        [pltpu.VMEM((CH, H), jnp.float32)] * NBUF,
        pltpu.VMEM((ZR, H), jnp.float32),
        pltpu.VMEM_SHARED((N, H), jnp.float32),
        [pltpu.SemaphoreType.DMA] * NBUF,
        [pltpu.SemaphoreType.DMA] * NBUF,
        [pltpu.SemaphoreType.DMA] * NBUF,
        pltpu.SemaphoreType.DMA,
    ]
    if with_counts:
        scratch += [
            pltpu.VMEM((CH, 16), jnp.float32),
            pltpu.VMEM((ZR, 16), jnp.float32),
            pltpu.VMEM_SHARED((N, 16), jnp.float32),
            pltpu.SemaphoreType.DMA,
        ]

    @functools.partial(
        pl.kernel,
        out_type=out_type,
        mesh=mesh,
        scratch_types=scratch,
        compiler_params=_SC_PARAMS,
    )
    def seg_sum(table_hbm, *rest):
        idx_refs, rest = rest[:2 * n_dirs], rest[2 * n_dirs:]
        if with_counts:
            (out_hbm, cnt_hbm, src_v, dst_v, rows_v, zero_v, acc_sh,
             ssem_v, dsem_v, gsem, wsem, ones_v, zero16_v, cnt_sh,
             csem) = rest
        else:
            (out_hbm, src_v, dst_v, rows_v, zero_v, acc_sh,
             ssem_v, dsem_v, gsem, wsem) = rest
        c = lax.axis_index("c")
        s = lax.axis_index("s")
        _fill_f32(zero_v, ZR, H, 0.0)
        if with_counts:
            _fill_f32(ones_v, CH, 16, 1.0)
            _fill_f32(zero16_v, ZR, 16, 0.0)

        def make_dir(d):
            src_hbm = idx_refs[2 * d]
            dst_hbm = idx_refs[2 * d + 1]
            off = c * cmul + doffs[d]

            def load_src(k, b):
                pltpu.async_copy(src_hbm.at[s, k], src_v[b], ssem_v[b])

            def wait_src(k, b):
                pltpu.make_async_copy(src_hbm.at[s, k], src_v[b],
                                      ssem_v[b]).wait()

            def load_dst(k, b):
                pltpu.async_copy(dst_hbm.at[s, k], dst_v[b], dsem_v[b])

            def wait_dst(k, b):
                pltpu.make_async_copy(dst_hbm.at[s, k], dst_v[b],
                                      dsem_v[b]).wait()

            def transform_src(b):
                for i in range(CH // 16):
                    src_v[b][pl.ds(i * 16, 16)] = (
                        src_v[b][pl.ds(i * 16, 16)] * mul + off)

            def start_gather(b):
                pltpu.async_copy(table_hbm.at[src_v[b]], rows_v[b],
                                 gsem[b])

            def wait_gather(b):
                pltpu.make_async_copy(table_hbm.at[src_v[b]], rows_v[b],
                                      gsem[b]).wait()

            def start_scatter(b):
                pltpu.async_copy(rows_v[b], acc_sh.at[dst_v[b]], wsem,
                                 add=True)

            def wait_scatter(b):
                pltpu.make_async_copy(rows_v[b], acc_sh.at[dst_v[b]],
                                      wsem).wait()

            def start_cnt(b):
                pltpu.async_copy(ones_v, cnt_sh.at[dst_v[b]], csem,
                                 add=True)

            def wait_cnt(b):
                pltpu.make_async_copy(ones_v, cnt_sh.at[dst_v[b]],
                                      csem).wait()

            for b in range(NBUF):
                load_src(b, b)
            for b in range(NBUF - 1):
                load_dst(b, b)
            for b in range(NBUF - 1):
                wait_src(b, b)
                transform_src(b)
                start_gather(b)
            _zero_acc(acc_sh, zero_v, s)
            if with_counts:
                @pl.when(c == d)
                def _zero_cnt():
                    _zero_acc(cnt_sh, zero16_v, s)
            plsc.subcore_barrier()

            def tri(t, inner):
                for b in range(NBUF):
                    j = NBUF * t + b
                    b2 = (b + NBUF - 1) % NBUF

                    @pl.when(j < ncd)
                    def _consume():
                        wait_gather(b)

                        @pl.when(j + NBUF < ncd)
                        def _prefetch_src():
                            load_src(j + NBUF, b)

                        @pl.when(j > 0)
                        def _drain_prev():
                            wait_scatter(b2)
                            if with_counts:
                                @pl.when(c == d)
                                def _drain_cnt():
                                    wait_cnt(b2)

                        wait_dst(j, b)
                        start_scatter(b)
                        if with_counts:
                            @pl.when(c == d)
                            def _cnt_scatter():
                                start_cnt(b)

                        @pl.when(j + NBUF - 1 < ncd)
                        def _prefetch_dst():
                            load_dst(j + NBUF - 1, b2)

                        @pl.when(j + NBUF - 1 < ncd)
                        def _issue_ahead():
                            wait_src(j + NBUF - 1, b2)
                            transform_src(b2)
                            start_gather(b2)
                return inner

            lax.fori_loop(0, (ncd + NBUF - 1) // NBUF, tri, None)
            wait_scatter((ncd - 1) % NBUF)
            if with_counts:
                @pl.when(c == d)
                def _tail_cnt():
                    wait_cnt((ncd - 1) % NBUF)
            plsc.subcore_barrier()

            pltpu.sync_copy(
                acc_sh.at[pl.ds(s * SLAB, SLAB)],
                out_hbm.at[d, c, pl.ds(s * SLAB, SLAB)])

            @pl.when(s == NS - 1)
            def _out_tail():
                pltpu.sync_copy(
                    acc_sh.at[pl.ds(NS * SLAB, N - NS * SLAB)],
                    out_hbm.at[d, c, pl.ds(NS * SLAB, N - NS * SLAB)])

            if with_counts:
                @pl.when(c == d)
                def _write_cnt():
                    pltpu.sync_copy(
                        cnt_sh.at[pl.ds(s * SLAB, SLAB)],
                        cnt_hbm.at[d, pl.ds(s * SLAB, SLAB)])

                    @pl.when(s == NS - 1)
                    def _cnt_tail():
                        pltpu.sync_copy(
                            cnt_sh.at[pl.ds(NS * SLAB, N - NS * SLAB)],
                            cnt_hbm.at[d, pl.ds(NS * SLAB, N - NS * SLAB)])

            plsc.subcore_barrier()

        for d in range(n_dirs):
            make_dir(d)

    return seg_sum


_RB = 1000
_NB = N // _RB


def _relu_body(x_ref, o_ref):
    o_ref[...] = jnp.maximum(x_ref[...], 0.0)


def _relu(x):
    n = x.shape[0]
    return pl.pallas_call(
        _relu_body,
        grid=(n // _RB,),
        in_specs=[pl.BlockSpec((_RB, D), lambda j: (j, 0))],
        out_specs=pl.BlockSpec((_RB, D), lambda j: (j, 0)),
        out_shape=jax.ShapeDtypeStruct((n, D), jnp.float32),
    )(x)


def _conv_pair_body(alo_ref, ahi_ref, cnt_ref, x_ref, w_ref, b_ref, o_ref):
    r = 1.0 / jnp.maximum(cnt_ref[0][:, 0:1], 1.0)
    a = jnp.concatenate([alo_ref[0, 0], ahi_ref[0, 0]], axis=1) * r
    o = (jnp.dot(a, w_ref[0, 0, :D], preferred_element_type=jnp.float32)
         + jnp.dot(x_ref[...], w_ref[0, 0, D:],
                   preferred_element_type=jnp.float32)
         + b_ref[0, 0])
    o_ref[0, 0] = jnp.maximum(o, 0.0)


def _conv_pair(agg, cnt, x_cat, w_cat, b_cat):
    return pl.pallas_call(
        _conv_pair_body,
        grid=(2, 2, _NB),
        in_specs=[
            pl.BlockSpec((1, 1, _RB, H), lambda d, h, j: (d, 0, j, 0)),
            pl.BlockSpec((1, 1, _RB, H), lambda d, h, j: (d, 1, j, 0)),
            pl.BlockSpec((1, _RB, 16), lambda d, h, j: (d, j, 0)),
            pl.BlockSpec((_RB, D), lambda d, h, j: ((1 - d) * _NB + j, 0)),
            pl.BlockSpec((1, 1, 2 * D, H), lambda d, h, j: (d, h, 0, 0)),
            pl.BlockSpec((1, 1, 1, H), lambda d, h, j: (d, h, 0, 0)),
        ],
        out_specs=pl.BlockSpec((1, 1, _RB, H), lambda d, h, j: (d, h, j, 0)),
        out_shape=jax.ShapeDtypeStruct((2, 2, N, H), jnp.float32),
    )(agg, agg, cnt, x_cat, w_cat, b_cat)


def _final_body(plo_ref, phi_ref, cnt_ref, xlo_ref, xhi_ref, w_ref, b_ref,
                wh_ref, bh_ref, o_ref):
    r = 1.0 / jnp.maximum(cnt_ref[0][:, 0:1], 1.0)
    a = jnp.concatenate([plo_ref[0, 0], phi_ref[0, 0]], axis=1) * r
    x = jnp.concatenate([xlo_ref[0, 0], xhi_ref[0, 0]], axis=1)
    o = (jnp.dot(a, w_ref[:D], preferred_element_type=jnp.float32)
         + jnp.dot(x, w_ref[D:], preferred_element_type=jnp.float32)
         + b_ref[...])
    o = jnp.maximum(o, 0.0)
    o_ref[...] = jnp.sum(o * wh_ref[...], axis=1, keepdims=True) + bh_ref[...]


def _final(agg2, cnts, x2, w_cat, b_cat, wh_row, bh):
    return pl.pallas_call(
        _final_body,
        grid=(_NB,),
        in_specs=[
            pl.BlockSpec((1, 1, _RB, H), lambda j: (0, 0, j, 0)),
            pl.BlockSpec((1, 1, _RB, H), lambda j: (0, 1, j, 0)),
            pl.BlockSpec((1, _RB, 16), lambda j: (0, j, 0)),
            pl.BlockSpec((1, 1, _RB, H), lambda j: (0, 0, j, 0)),
            pl.BlockSpec((1, 1, _RB, H), lambda j: (0, 1, j, 0)),
            pl.BlockSpec((2 * D, D), lambda j: (0, 0)),
            pl.BlockSpec((1, D), lambda j: (0, 0)),
            pl.BlockSpec((1, D), lambda j: (0, 0)),
            pl.BlockSpec((1, 1), lambda j: (0, 0)),
        ],
        out_specs=pl.BlockSpec((_RB, 1), lambda j: (j, 0)),
        out_shape=jax.ShapeDtypeStruct((N, 1), jnp.float32),
    )(agg2, agg2, cnts, x2, x2, w_cat, b_cat, wh_row, bh)


def kernel(u2i_src, u2i_dst, i2u_src, i2u_dst, emb_user, emb_item,
           W1un, b1un, W1us, b1us, W1in, b1in, W1is, b1is,
           W2un, b2un, W2us, b2us, W2in, b2in, W2is, b2is, W_head, b_head):
    ncd = E // (NS * CH)

    def chunked(idx):
        return idx.reshape(NS, ncd, CH).astype(jnp.int32)

    su, du = chunked(u2i_src), chunked(u2i_dst)
    si, di = chunked(i2u_src), chunked(i2u_dst)

    x_cat = _relu(jnp.concatenate([emb_user, emb_item], axis=0))

    agg1, cnts = _make_seg_sum(2, ncd, 2, (1, (0, 2 * N)),
                               with_counts=True)(
        x_cat.reshape(4 * N, H), su, du, si, di)

    w1 = jnp.stack([jnp.concatenate([W1un, W1us], axis=0),
                    jnp.concatenate([W1in, W1is], axis=0)])
    w1 = w1.reshape(2, 2 * D, 2, H).transpose(0, 2, 1, 3)
    b1 = jnp.stack([(b1un + b1us), (b1in + b1is)]).reshape(2, 2, 1, H)
    x2 = _conv_pair(agg1, cnts, x_cat, w1, b1)

    agg2 = _make_seg_sum(1, ncd, 1, (N, (2 * N,)))(
        x2.reshape(4 * N, H), su, du)

    w2 = jnp.concatenate([W2un, W2us], axis=0)
    b2 = (b2un + b2us)[None, :]
    out = _final(agg2, cnts, x2, w2, b2, W_head.T, b_head[None, :])
    return out[:, 0]
